# baseline (speedup 1.0000x reference)
import functools

import jax
import jax.numpy as jnp
from jax import lax
from jax.experimental import pallas as pl
from jax.experimental.pallas import tpu as pltpu
from jax.experimental.pallas import tpu_sc as plsc

N_NODES = 100000
D = 128
S = 256
B = 128
NW = 32
NB = 12
SC_ROWS = NW * NB * B
TC_ROWS = N_NODES - SC_ROWS
TCB = 2048
TC_BLOCKS = -(-TC_ROWS // TCB)
TC_PAD = TC_BLOCKS * TCB

_mesh = plsc.VectorSubcoreMesh(core_axis_name="c", subcore_axis_name="s")


@functools.partial(
    pl.kernel,
    out_type=jax.ShapeDtypeStruct((2, S, D), jnp.float32),
    mesh=_mesh,
    scratch_types=[
        pltpu.VMEM((B,), jnp.int32),
        pltpu.VMEM((B,), jnp.int32),
        pltpu.VMEM((B, D), jnp.float32),
        pltpu.VMEM((B, D), jnp.float32),
        pltpu.VMEM((16, D), jnp.float32),
        pltpu.VMEM_SHARED((S, D), jnp.float32),
        pltpu.SemaphoreType.DMA,
        pltpu.SemaphoreType.DMA,
        pltpu.SemaphoreType.DMA,
        pltpu.SemaphoreType.DMA,
    ],
)
def _sc_segsum(feat_hbm, ids_hbm, out_hbm, idx0, idx1, rows0, rows1,
               stage_v, acc_sh, dsem0, dsem1, ssem0, ssem1):
    cid = lax.axis_index("c")
    sid = lax.axis_index("s")
    wid = sid * 2 + cid

    rows = (rows0, rows1)
    idx = (idx0, idx1)
    dsem = (dsem0, dsem1)
    ssem = (ssem0, ssem1)

    zeros16 = jnp.zeros((16,), jnp.float32)
    for r in range(16):
        for c in range(D // 16):
            stage_v[r, pl.ds(c * 16, 16)] = zeros16
    pltpu.sync_copy(stage_v, acc_sh.at[pl.ds(sid * 16, 16)])
    plsc.subcore_barrier()

    row0 = wid * NB * B

    def start(j):
        pltpu.async_copy(ids_hbm.at[pl.ds(row0 + j * B, B)], idx[j % 2],
                         dsem[j % 2])
        pltpu.async_copy(feat_hbm.at[pl.ds(row0 + j * B, B)], rows[j % 2],
                         dsem[j % 2])

    def wait_rows(j):
        pltpu.make_async_copy(ids_hbm.at[pl.ds(row0 + j * B, B)],
                              idx[j % 2], dsem[j % 2]).wait()
        pltpu.make_async_copy(feat_hbm.at[pl.ds(row0 + j * B, B)],
                              rows[j % 2], dsem[j % 2]).wait()

    def scat(j):
        pltpu.async_copy(rows[j % 2], acc_sh.at[idx[j % 2]], ssem[j % 2],
                         add=True)

    def wait_scat(j):
        pltpu.make_async_copy(rows[j % 2], acc_sh.at[idx[j % 2]],
                              ssem[j % 2]).wait()

    start(0)
    for i in range(NB):
        if i + 1 < NB:
            if i - 1 >= 0:
                wait_scat(i - 1)
            start(i + 1)
        wait_rows(i)
        scat(i)
    wait_scat(NB - 2)
    wait_scat(NB - 1)

    plsc.subcore_barrier()

    pltpu.sync_copy(acc_sh.at[pl.ds(sid * 16, 16)], stage_v)
    pltpu.sync_copy(stage_v, out_hbm.at[cid, pl.ds(sid * 16, 16)])


def _tc_body(ids_ref, feat_ref, o_ref):
    i = pl.program_id(0)
    ids_blk = ids_ref[0, 0]
    onehot = (lax.broadcasted_iota(jnp.int32, (S, TCB), 0)
              == ids_blk[None, :]).astype(jnp.float32)
    partial = jax.lax.dot(onehot, feat_ref[...],
                          precision=lax.Precision.HIGHEST,
                          preferred_element_type=jnp.float32)

    @pl.when(i == 0)
    def _():
        o_ref[...] = jnp.zeros_like(o_ref)

    o_ref[...] += partial


def kernel(features, segment_ids):
    ids = segment_ids.astype(jnp.int32)

    partials = _sc_segsum(features, ids)

    ids_hi = jnp.full((TC_PAD,), S, jnp.int32).at[:TC_ROWS].set(ids[SC_ROWS:])
    tc_out = pl.pallas_call(
        _tc_body,
        grid=(TC_BLOCKS,),
        in_specs=[
            pl.BlockSpec((1, 1, TCB), lambda i: (i, 0, 0)),
            pl.BlockSpec((TCB, D), lambda i: (i + SC_ROWS // TCB, 0)),
        ],
        out_specs=pl.BlockSpec((S, D), lambda i: (0, 0)),
        out_shape=jax.ShapeDtypeStruct((S, D), jnp.float32),
    )(ids_hi.reshape(TC_BLOCKS, 1, TCB), features)

    def _combine_body(p_ref, t_ref, o_ref):
        o_ref[...] = p_ref[0] + p_ref[1] + t_ref[...]

    return pl.pallas_call(
        _combine_body,
        out_shape=jax.ShapeDtypeStruct((S, D), jnp.float32),
    )(partials, tc_out)


# speedup vs baseline: 1.3844x; 1.3844x over previous
import functools

import jax
import jax.numpy as jnp
from jax import lax
from jax.experimental import pallas as pl
from jax.experimental.pallas import tpu as pltpu
from jax.experimental.pallas import tpu_sc as plsc

N_NODES = 100000
D = 128
S = 256
B = 128
NW = 32
MAXNB = 25
NB31 = 6
TAIL = 32
TAIL_BASE = N_NODES - TAIL

_mesh = plsc.VectorSubcoreMesh(core_axis_name="c", subcore_axis_name="s")


@functools.partial(
    pl.kernel,
    out_type=jax.ShapeDtypeStruct((2, S, D), jnp.float32),
    mesh=_mesh,
    scratch_types=[
        pltpu.VMEM((4, B), jnp.int32),
        pltpu.VMEM((B, D), jnp.float32),
        pltpu.VMEM((B, D), jnp.float32),
        pltpu.VMEM((B, D), jnp.float32),
        pltpu.VMEM((B, D), jnp.float32),
        pltpu.VMEM((TAIL,), jnp.int32),
        pltpu.VMEM((TAIL, D), jnp.float32),
        pltpu.VMEM((16, D), jnp.float32),
        pltpu.VMEM_SHARED((S, D), jnp.float32),
        pltpu.SemaphoreType.DMA,
        pltpu.SemaphoreType.DMA,
        pltpu.SemaphoreType.DMA,
        pltpu.SemaphoreType.DMA,
        pltpu.SemaphoreType.DMA,
        pltpu.SemaphoreType.DMA,
        pltpu.SemaphoreType.DMA,
        pltpu.SemaphoreType.DMA,
    ],
)
def _sc_segsum(feat_hbm, ids_hbm, out_hbm, idxb, rows0, rows1, rows2, rows3,
               tidx_v, trows_v, stage_v, acc_sh,
               dsem0, dsem1, dsem2, dsem3, ssem0, ssem1, ssem2, ssem3):
    cid = lax.axis_index("c")
    sid = lax.axis_index("s")
    wid = sid * 2 + cid

    rows = (rows0, rows1, rows2, rows3)
    dsem = (dsem0, dsem1, dsem2, dsem3)
    ssem = (ssem0, ssem1, ssem2, ssem3)
    NBUF = 4

    zeros16 = jnp.zeros((16,), jnp.float32)
    for r in range(16):
        for c in range(D // 16):
            stage_v[r, pl.ds(c * 16, 16)] = zeros16
    pltpu.sync_copy(stage_v, acc_sh.at[pl.ds(sid * 16, 16)])
    plsc.subcore_barrier()

    row0 = wid * MAXNB * B

    def guard(j):
        return (wid < NW - 1) | (j < NB31)

    def start(j):
        pltpu.async_copy(ids_hbm.at[pl.ds(row0 + j * B, B)],
                         idxb.at[j % NBUF], dsem[j % NBUF])
        pltpu.async_copy(feat_hbm.at[pl.ds(row0 + j * B, B)], rows[j % NBUF],
                         dsem[j % NBUF])

    def wait_rows(j):
        pltpu.make_async_copy(ids_hbm.at[pl.ds(row0 + j * B, B)],
                              idxb.at[j % NBUF], dsem[j % NBUF]).wait()
        pltpu.make_async_copy(feat_hbm.at[pl.ds(row0 + j * B, B)],
                              rows[j % NBUF], dsem[j % NBUF]).wait()

    def scat(j):
        pltpu.async_copy(rows[j % NBUF], acc_sh.at[idxb.at[j % NBUF]],
                         ssem[j % NBUF], add=True)

    def wait_scat(j):
        pltpu.make_async_copy(rows[j % NBUF], acc_sh.at[idxb.at[j % NBUF]],
                              ssem[j % NBUF]).wait()

    def maybe(j, fn):
        if j < NB31:
            fn(j)
        else:
            pl.when(guard(j))(lambda: fn(j))

    for j in range(3):
        maybe(j, start)
    for i in range(MAXNB):
        if i + 3 < MAXNB:
            if i - 1 >= 0:
                maybe(i - 1, wait_scat)
            maybe(i + 3, start)
        maybe(i, wait_rows)
        maybe(i, scat)
    for j in range(MAXNB - 4, MAXNB):
        maybe(j, wait_scat)

    @pl.when(wid == NW - 1)
    def _():
        pltpu.sync_copy(ids_hbm.at[pl.ds(TAIL_BASE, TAIL)], tidx_v)
        pltpu.sync_copy(feat_hbm.at[pl.ds(TAIL_BASE, TAIL)], trows_v)
        pltpu.sync_copy(trows_v, acc_sh.at[tidx_v], add=True)

    plsc.subcore_barrier()

    pltpu.sync_copy(acc_sh.at[pl.ds(sid * 16, 16)],
                    out_hbm.at[cid, pl.ds(sid * 16, 16)])


def _combine_body(p_ref, o_ref):
    o_ref[...] = p_ref[0] + p_ref[1]


def kernel(features, segment_ids):
    ids = segment_ids.astype(jnp.int32)
    partials = _sc_segsum(features, ids)
    return pl.pallas_call(
        _combine_body,
        out_shape=jax.ShapeDtypeStruct((S, D), jnp.float32),
    )(partials)
